# baseline (speedup 1.0000x reference)
import functools

import jax
import jax.numpy as jnp
from jax import lax
from jax.experimental import pallas as pl
from jax.experimental.pallas import tpu as pltpu
from jax.experimental.pallas import tpu_sc as plsc

_N = 8192
_XD = 2048
_CD = 128
_HD = 64
_E = 64
_TN = 512
_GRID = _N // _TN
_LOSS_COEF = 1e-2

_NW = 32
_TPW = _N // _NW
_L = 16


def _enc_body(x_ref, ew0, eb0, ew1, eb1, ew2, eb2, wg,
              z_ref, lt_ref):
    f32 = jnp.float32
    h0_t = lax.dot_general(ew0[...], x_ref[...],
                           dimension_numbers=(((0,), (1,)), ((), ())),
                           preferred_element_type=f32)
    h = jax.nn.relu(h0_t.T + eb0[...])
    h = jax.nn.relu(jnp.dot(h, ew1[...], preferred_element_type=f32) + eb1[...])
    z = jnp.dot(h, ew2[...], preferred_element_type=f32) + eb2[...]
    z_ref[...] = z
    lt_ref[...] = lax.dot_general(wg[...], z,
                                  dimension_numbers=(((0,), (1,)), ((), ())),
                                  preferred_element_type=f32)


def _encoder(x, enc_W0, enc_b0, enc_W1, enc_b1, enc_W2, enc_b2, w_gate):
    full = lambda shp: pl.BlockSpec(shp, lambda i: (0, 0))
    return pl.pallas_call(
        _enc_body,
        grid=(_GRID,),
        in_specs=[
            pl.BlockSpec((_TN, _XD), lambda i: (i, 0)),
            full((_XD, _HD)), full((1, _HD)), full((_HD, _HD)), full((1, _HD)),
            full((_HD, _HD)), full((1, _HD)), full((_HD, _E)),
        ],
        out_specs=(
            pl.BlockSpec((_TN, _HD), lambda i: (i, 0)),
            pl.BlockSpec((_E, _TN), lambda i: (0, i)),
        ),
        out_shape=(
            jax.ShapeDtypeStruct((_N, _HD), jnp.float32),
            jax.ShapeDtypeStruct((_E, _N), jnp.float32),
        ),
    )(x, enc_W0, enc_b0.reshape(1, _HD), enc_W1, enc_b1.reshape(1, _HD),
      enc_W2, enc_b2.reshape(1, _HD), w_gate)


def _gate_sc_body(lt_hbm, gates_hbm, lt_v, g_v):
    f32 = jnp.float32
    i32 = jnp.int32
    wid = lax.axis_index("s") * 2 + lax.axis_index("c")
    base = wid * _TPW
    pltpu.sync_copy(lt_hbm.at[:, pl.ds(base, _TPW)], lt_v)

    def _group(g, _):
        m1 = jnp.full((_L,), -jnp.inf, f32)
        m2 = jnp.full((_L,), -jnp.inf, f32)
        i1 = jnp.zeros((_L,), i32)
        i2 = jnp.zeros((_L,), i32)
        col = g * _L
        for e in range(_E):
            v = lt_v[e, pl.ds(col, _L)]
            upd1 = v > m1
            upd2 = v > m2
            i2 = jnp.where(upd1, i1, jnp.where(upd2, e, i2))
            m2 = jnp.where(upd1, m1, jnp.where(upd2, v, m2))
            i1 = jnp.where(upd1, e, i1)
            m1 = jnp.where(upd1, v, m1)
        e2 = jnp.exp(m2 - m1)
        den = 1.0 + e2
        g1 = 1.0 / den
        g2 = e2 / den
        zero = jnp.zeros((_L,), f32)
        for e in range(_E):
            val = (jnp.where(i1 == e, g1, zero) +
                   jnp.where(i2 == e, g2, zero))
            g_v[e, pl.ds(col, _L)] = val
        return _
    lax.fori_loop(0, _TPW // _L, _group, 0)

    pltpu.sync_copy(g_v, gates_hbm.at[:, pl.ds(base, _TPW)])


def _gating(logits_t):
    mesh = plsc.VectorSubcoreMesh(core_axis_name="c", subcore_axis_name="s")
    k = functools.partial(
        pl.kernel,
        mesh=mesh,
        out_type=jax.ShapeDtypeStruct((_E, _N), jnp.float32),
        scratch_types=[
            pltpu.VMEM((_E, _TPW), jnp.float32),
            pltpu.VMEM((_E, _TPW), jnp.float32),
        ],
    )(_gate_sc_body)
    return k(logits_t)


def _dec_body(z_ref, dw0, db0, dw1, db1, dw2, db2, xhat_ref):
    f32 = jnp.float32
    bf16 = jnp.bfloat16
    z = z_ref[...]
    h = jax.nn.relu(jnp.dot(z, dw0[...], preferred_element_type=f32) + db0[...])
    h = jax.nn.relu(jnp.dot(h, dw1[...], preferred_element_type=f32) + db1[...])
    xhat_ref[...] = (jnp.dot(h.astype(bf16), dw2[...], preferred_element_type=f32)
                     + db2[...])


def _decoder(z, dec_W0, dec_b0, dec_W1, dec_b1, dec_W2b, dec_b2):
    full = lambda shp: pl.BlockSpec(shp, lambda i: (0, 0))
    return pl.pallas_call(
        _dec_body,
        grid=(_GRID,),
        in_specs=[
            pl.BlockSpec((_TN, _HD), lambda i: (i, 0)),
            full((_HD, _HD)), full((1, _HD)), full((_HD, _HD)), full((1, _HD)),
            full((_HD, _XD)), full((1, _XD)),
        ],
        out_specs=pl.BlockSpec((_TN, _XD), lambda i: (i, 0)),
        out_shape=jax.ShapeDtypeStruct((_N, _XD), jnp.float32),
    )(z, dec_W0, dec_b0.reshape(1, _HD), dec_W1, dec_b1.reshape(1, _HD),
      dec_W2b, dec_b2.reshape(1, _XD))


def _exp_body(z_ref, gt_ref, c_ref, w1cat, b1cat, w2v, eb2x, expand_ref,
              w_ref, prob_ref, loss_ref, gates_ref, acc_ref):
    i = pl.program_id(0)
    f32 = jnp.float32
    bf16 = jnp.bfloat16
    z = z_ref[...]
    gates = gt_ref[...].T
    gates_ref[...] = gates

    hh = jax.nn.relu(jnp.dot(z.astype(bf16), w1cat[...],
                             preferred_element_type=f32) +
                     b1cat[...]).astype(bf16)
    gx = jnp.dot(gates.astype(bf16), expand_ref[...],
                 preferred_element_type=f32).astype(bf16)
    wout = (jnp.dot(hh * gx, w2v[...], preferred_element_type=f32) +
            jnp.dot(gates, eb2x[...], preferred_element_type=f32))
    w_ref[...] = wout

    score = jnp.sum(c_ref[...] * wout, axis=1, keepdims=True)
    prob_ref[...] = 1.0 / (1.0 + jnp.exp(-score))

    @pl.when(i == 0)
    def _init():
        acc_ref[...] = jnp.zeros_like(acc_ref)

    acc_ref[0:1, :] = acc_ref[0:1, :] + jnp.sum(gates, axis=0, keepdims=True)
    acc_ref[1:2, :] = acc_ref[1:2, :] + jnp.sum((gates > 0).astype(f32), axis=0,
                                                keepdims=True)

    @pl.when(i == _GRID - 1)
    def _loss():
        def cv2(v):
            mean = jnp.sum(v) / _E
            var = jnp.sum((v - mean) ** 2) / (_E - 1)
            return var / (mean * mean + 1e-10)
        loss_ref[0, 0] = (cv2(acc_ref[0:1, :]) + cv2(acc_ref[1:2, :])) * _LOSS_COEF


def _experts(z, gates_t, c, w1cat, b1cat, w2v, e_b2, expand):
    full = lambda shp: pl.BlockSpec(shp, lambda i: (0, 0))
    return pl.pallas_call(
        _exp_body,
        grid=(_GRID,),
        in_specs=[
            pl.BlockSpec((_TN, _HD), lambda i: (i, 0)),
            pl.BlockSpec((_E, _TN), lambda i: (0, i)),
            pl.BlockSpec((_TN, _CD), lambda i: (i, 0)),
            full((_HD, _E * _HD)), full((1, _E * _HD)),
            full((_E * _HD, _CD)), full((_E, _CD)), full((_E, _E * _HD)),
        ],
        out_specs=(
            pl.BlockSpec((_TN, _CD), lambda i: (i, 0)),
            pl.BlockSpec((_TN, 1), lambda i: (i, 0)),
            pl.BlockSpec((1, 1), lambda i: (0, 0), memory_space=pltpu.SMEM),
            pl.BlockSpec((_TN, _E), lambda i: (i, 0)),
        ),
        out_shape=(
            jax.ShapeDtypeStruct((_N, _CD), jnp.float32),
            jax.ShapeDtypeStruct((_N, 1), jnp.float32),
            jax.ShapeDtypeStruct((1, 1), jnp.float32),
            jax.ShapeDtypeStruct((_N, _E), jnp.float32),
        ),
        scratch_shapes=[pltpu.VMEM((8, _E), jnp.float32)],
    )(z, gates_t, c, w1cat, b1cat, w2v, e_b2, expand)


def kernel(x, c, enc_W0, enc_b0, enc_W1, enc_b1, enc_W2, enc_b2,
           dec_W0, dec_b0, dec_W1, dec_b1, dec_W2, dec_b2,
           w_gate, e_W1, e_b1, e_W2, e_b2):
    bf16 = jnp.bfloat16
    w1cat = jnp.transpose(e_W1, (1, 0, 2)).reshape(_HD, _E * _HD).astype(bf16)
    b1cat = e_b1.reshape(1, _E * _HD)
    w2v = e_W2.reshape(_E * _HD, _CD).astype(bf16)
    expand = jnp.kron(jnp.eye(_E, dtype=bf16), jnp.ones((1, _HD), dtype=bf16))

    z, logits_t = _encoder(x, enc_W0, enc_b0, enc_W1, enc_b1, enc_W2, enc_b2,
                           w_gate)
    gates_t = _gating(logits_t)
    xhat = _decoder(z, dec_W0, dec_b0, dec_W1, dec_b1, dec_W2.astype(bf16),
                    dec_b2)
    w, prob, loss, gates = _experts(z, gates_t, c, w1cat, b1cat, w2v, e_b2,
                                    expand)
    return (w, prob, z, xhat, loss.reshape(()), gates)


# speedup vs baseline: 1.0916x; 1.0916x over previous
import functools

import jax
import jax.numpy as jnp
from jax import lax
from jax.experimental import pallas as pl
from jax.experimental.pallas import tpu as pltpu
from jax.experimental.pallas import tpu_sc as plsc

_N = 8192
_XD = 2048
_CD = 128
_HD = 64
_E = 64
_TN = 512
_GRID = _N // _TN
_LOSS_COEF = 1e-2

_NW = 32
_TPW = _N // _NW
_L = 16


def _enc_body(x_ref, ew0, eb0, ew1, eb1, ew2, eb2, wg,
              z_ref, lt_ref):
    f32 = jnp.float32
    h0_t = lax.dot_general(ew0[...], x_ref[...],
                           dimension_numbers=(((0,), (1,)), ((), ())),
                           preferred_element_type=f32)
    h = jax.nn.relu(h0_t.T + eb0[...])
    h = jax.nn.relu(jnp.dot(h, ew1[...], preferred_element_type=f32) + eb1[...])
    z = jnp.dot(h, ew2[...], preferred_element_type=f32) + eb2[...]
    z_ref[...] = z
    lt_ref[...] = lax.dot_general(wg[...], z,
                                  dimension_numbers=(((0,), (1,)), ((), ())),
                                  preferred_element_type=f32)


def _encoder(x, enc_W0, enc_b0, enc_W1, enc_b1, enc_W2, enc_b2, w_gate):
    full = lambda shp: pl.BlockSpec(shp, lambda i: (0, 0))
    return pl.pallas_call(
        _enc_body,
        grid=(_GRID,),
        in_specs=[
            pl.BlockSpec((_TN, _XD), lambda i: (i, 0)),
            full((_XD, _HD)), full((1, _HD)), full((_HD, _HD)), full((1, _HD)),
            full((_HD, _HD)), full((1, _HD)), full((_HD, _E)),
        ],
        out_specs=(
            pl.BlockSpec((_TN, _HD), lambda i: (i, 0)),
            pl.BlockSpec((_E, _TN), lambda i: (0, i)),
        ),
        out_shape=(
            jax.ShapeDtypeStruct((_N, _HD), jnp.float32),
            jax.ShapeDtypeStruct((_E, _N), jnp.float32),
        ),
    )(x, enc_W0, enc_b0.reshape(1, _HD), enc_W1, enc_b1.reshape(1, _HD),
      enc_W2, enc_b2.reshape(1, _HD), w_gate)


def _gate_sc_body(lt_hbm, gates_hbm, lt_v, g_v):
    f32 = jnp.float32
    i32 = jnp.int32
    wid = lax.axis_index("s") * 2 + lax.axis_index("c")
    base = wid * _TPW
    pltpu.sync_copy(lt_hbm.at[:, pl.ds(base, _TPW)], lt_v)

    def _group(g, _):
        m1 = jnp.full((_L,), -jnp.inf, f32)
        m2 = jnp.full((_L,), -jnp.inf, f32)
        i1 = jnp.zeros((_L,), i32)
        i2 = jnp.zeros((_L,), i32)
        col = g * _L
        for e in range(_E):
            v = lt_v[e, pl.ds(col, _L)]
            upd1 = v > m1
            upd2 = v > m2
            i2 = jnp.where(upd1, i1, jnp.where(upd2, e, i2))
            m2 = jnp.where(upd1, m1, jnp.where(upd2, v, m2))
            i1 = jnp.where(upd1, e, i1)
            m1 = jnp.where(upd1, v, m1)
        e2 = jnp.exp(m2 - m1)
        den = 1.0 + e2
        g1 = 1.0 / den
        g2 = e2 / den
        zero = jnp.zeros((_L,), f32)
        for e in range(_E):
            val = (jnp.where(i1 == e, g1, zero) +
                   jnp.where(i2 == e, g2, zero))
            g_v[e, pl.ds(col, _L)] = val
        return _
    lax.fori_loop(0, _TPW // _L, _group, 0)

    pltpu.sync_copy(g_v, gates_hbm.at[:, pl.ds(base, _TPW)])


def _gating(logits_t):
    mesh = plsc.VectorSubcoreMesh(core_axis_name="c", subcore_axis_name="s")
    k = functools.partial(
        pl.kernel,
        mesh=mesh,
        out_type=jax.ShapeDtypeStruct((_E, _N), jnp.float32),
        scratch_types=[
            pltpu.VMEM((_E, _TPW), jnp.float32),
            pltpu.VMEM((_E, _TPW), jnp.float32),
        ],
    )(_gate_sc_body)
    return k(logits_t)


def _dec_body(z_ref, dw0, db0, dw1, db1, dw2, db2, xhat_ref):
    f32 = jnp.float32
    bf16 = jnp.bfloat16
    z = z_ref[...]
    h = jax.nn.relu(jnp.dot(z, dw0[...], preferred_element_type=f32) + db0[...])
    h = jax.nn.relu(jnp.dot(h, dw1[...], preferred_element_type=f32) + db1[...])
    xhat_ref[...] = (jnp.dot(h.astype(bf16), dw2[...], preferred_element_type=f32)
                     + db2[...])


def _decoder(z, dec_W0, dec_b0, dec_W1, dec_b1, dec_W2b, dec_b2):
    full = lambda shp: pl.BlockSpec(shp, lambda i: (0, 0))
    return pl.pallas_call(
        _dec_body,
        grid=(_GRID,),
        in_specs=[
            pl.BlockSpec((_TN, _HD), lambda i: (i, 0)),
            full((_HD, _HD)), full((1, _HD)), full((_HD, _HD)), full((1, _HD)),
            full((_HD, _XD)), full((1, _XD)),
        ],
        out_specs=pl.BlockSpec((_TN, _XD), lambda i: (i, 0)),
        out_shape=jax.ShapeDtypeStruct((_N, _XD), jnp.float32),
    )(z, dec_W0, dec_b0.reshape(1, _HD), dec_W1, dec_b1.reshape(1, _HD),
      dec_W2b, dec_b2.reshape(1, _XD))


def _exp_body(z_ref, gt_ref, c_ref, dw0, db0, dw1, db1, dw2, db2,
              w1cat, b1cat, w2v, eb2x, expand_ref,
              w_ref, prob_ref, loss_ref, gates_ref, xhat_ref, acc_ref):
    i = pl.program_id(0)
    f32 = jnp.float32
    bf16 = jnp.bfloat16
    z = z_ref[...]
    gates = gt_ref[...].T
    gates_ref[...] = gates

    h = jax.nn.relu(jnp.dot(z, dw0[...], preferred_element_type=f32) + db0[...])
    h = jax.nn.relu(jnp.dot(h, dw1[...], preferred_element_type=f32) + db1[...])
    xhat_ref[...] = (jnp.dot(h.astype(bf16), dw2[...], preferred_element_type=f32)
                     + db2[...])

    hh = jax.nn.relu(jnp.dot(z.astype(bf16), w1cat[...],
                             preferred_element_type=f32) +
                     b1cat[...]).astype(bf16)
    gx = jnp.dot(gates.astype(bf16), expand_ref[...],
                 preferred_element_type=f32).astype(bf16)
    wout = (jnp.dot(hh * gx, w2v[...], preferred_element_type=f32) +
            jnp.dot(gates, eb2x[...], preferred_element_type=f32))
    w_ref[...] = wout

    score = jnp.sum(c_ref[...] * wout, axis=1, keepdims=True)
    prob_ref[...] = 1.0 / (1.0 + jnp.exp(-score))

    @pl.when(i == 0)
    def _init():
        acc_ref[...] = jnp.zeros_like(acc_ref)

    acc_ref[0:1, :] = acc_ref[0:1, :] + jnp.sum(gates, axis=0, keepdims=True)
    acc_ref[1:2, :] = acc_ref[1:2, :] + jnp.sum((gates > 0).astype(f32), axis=0,
                                                keepdims=True)

    @pl.when(i == _GRID - 1)
    def _loss():
        def cv2(v):
            mean = jnp.sum(v) / _E
            var = jnp.sum((v - mean) ** 2) / (_E - 1)
            return var / (mean * mean + 1e-10)
        loss_ref[0, 0] = (cv2(acc_ref[0:1, :]) + cv2(acc_ref[1:2, :])) * _LOSS_COEF


def _experts(z, gates_t, c, dec_W0, dec_b0, dec_W1, dec_b1, dec_W2b, dec_b2,
             w1cat, b1cat, w2v, e_b2, expand):
    full = lambda shp: pl.BlockSpec(shp, lambda i: (0, 0))
    return pl.pallas_call(
        _exp_body,
        grid=(_GRID,),
        in_specs=[
            pl.BlockSpec((_TN, _HD), lambda i: (i, 0)),
            pl.BlockSpec((_E, _TN), lambda i: (0, i)),
            pl.BlockSpec((_TN, _CD), lambda i: (i, 0)),
            full((_HD, _HD)), full((1, _HD)), full((_HD, _HD)), full((1, _HD)),
            full((_HD, _XD)), full((1, _XD)),
            full((_HD, _E * _HD)), full((1, _E * _HD)),
            full((_E * _HD, _CD)), full((_E, _CD)), full((_E, _E * _HD)),
        ],
        out_specs=(
            pl.BlockSpec((_TN, _CD), lambda i: (i, 0)),
            pl.BlockSpec((_TN, 1), lambda i: (i, 0)),
            pl.BlockSpec((1, 1), lambda i: (0, 0), memory_space=pltpu.SMEM),
            pl.BlockSpec((_TN, _E), lambda i: (i, 0)),
            pl.BlockSpec((_TN, _XD), lambda i: (i, 0)),
        ),
        out_shape=(
            jax.ShapeDtypeStruct((_N, _CD), jnp.float32),
            jax.ShapeDtypeStruct((_N, 1), jnp.float32),
            jax.ShapeDtypeStruct((1, 1), jnp.float32),
            jax.ShapeDtypeStruct((_N, _E), jnp.float32),
            jax.ShapeDtypeStruct((_N, _XD), jnp.float32),
        ),
        scratch_shapes=[pltpu.VMEM((8, _E), jnp.float32)],
    )(z, gates_t, c, dec_W0, dec_b0.reshape(1, _HD), dec_W1,
      dec_b1.reshape(1, _HD), dec_W2b, dec_b2.reshape(1, _XD),
      w1cat, b1cat, w2v, e_b2, expand)


def kernel(x, c, enc_W0, enc_b0, enc_W1, enc_b1, enc_W2, enc_b2,
           dec_W0, dec_b0, dec_W1, dec_b1, dec_W2, dec_b2,
           w_gate, e_W1, e_b1, e_W2, e_b2):
    bf16 = jnp.bfloat16
    w1cat = jnp.transpose(e_W1, (1, 0, 2)).reshape(_HD, _E * _HD).astype(bf16)
    b1cat = e_b1.reshape(1, _E * _HD)
    w2v = e_W2.reshape(_E * _HD, _CD).astype(bf16)
    expand = jnp.kron(jnp.eye(_E, dtype=bf16), jnp.ones((1, _HD), dtype=bf16))

    z, logits_t = _encoder(x, enc_W0, enc_b0, enc_W1, enc_b1, enc_W2, enc_b2,
                           w_gate)
    gates_t = _gating(logits_t)
    w, prob, loss, gates, xhat = _experts(
        z, gates_t, c, dec_W0, dec_b0, dec_W1, dec_b1, dec_W2.astype(bf16),
        dec_b2, w1cat, b1cat, w2v, e_b2, expand)
    return (w, prob, z, xhat, loss.reshape(()), gates)


# speedup vs baseline: 1.1644x; 1.0667x over previous
import functools

import jax
import jax.numpy as jnp
from jax import lax
from jax.experimental import pallas as pl
from jax.experimental.pallas import tpu as pltpu
from jax.experimental.pallas import tpu_sc as plsc

_N = 8192
_XD = 2048
_CD = 128
_HD = 64
_E = 64
_TN = 512
_GRID = _N // _TN
_LOSS_COEF = 1e-2

_NW = 32
_TPW = _N // _NW
_L = 16


def _body(x_ref, c_ref,
          ew0, eb0, ew1, eb1, ew2, eb2,
          dw0, db0, dw1, db1, dw2, db2,
          wg, w1cat, b1cat, w2v, eb2x, eio_ref, expand_ref,
          w_ref, prob_ref, z_ref, xhat_ref, gates_ref):
    f32 = jnp.float32
    bf16 = jnp.bfloat16

    h0_t = lax.dot_general(ew0[...], x_ref[...],
                           dimension_numbers=(((0,), (1,)), ((), ())),
                           preferred_element_type=f32)
    h = jax.nn.relu(h0_t.T + eb0[...])
    h = jax.nn.relu(jnp.dot(h, ew1[...], preferred_element_type=f32) + eb1[...])
    z = jnp.dot(h, ew2[...], preferred_element_type=f32) + eb2[...]
    z_ref[...] = z

    h = jax.nn.relu(jnp.dot(z, dw0[...], preferred_element_type=f32) + db0[...])
    h = jax.nn.relu(jnp.dot(h, dw1[...], preferred_element_type=f32) + db1[...])
    xhat_ref[...] = (jnp.dot(h.astype(bf16), dw2[...],
                             preferred_element_type=f32) + db2[...])

    logits = jnp.dot(z, wg[...], preferred_element_type=f32)
    eio = eio_ref[...]
    m1 = jnp.max(logits, axis=1, keepdims=True)
    i1 = jnp.min(jnp.where(logits == m1, eio, _E), axis=1, keepdims=True)
    masked = jnp.where(eio == i1, -jnp.inf, logits)
    m2 = jnp.max(masked, axis=1, keepdims=True)
    i2 = jnp.min(jnp.where(masked == m2, eio, _E), axis=1, keepdims=True)
    e2 = jnp.exp(m2 - m1)
    denom = 1.0 + e2
    g1 = 1.0 / denom
    g2 = e2 / denom
    gates = jnp.where(eio == i1, g1, 0.0) + jnp.where(eio == i2, g2, 0.0)
    gates_ref[...] = gates

    hh = jax.nn.relu(jnp.dot(z.astype(bf16), w1cat[...],
                             preferred_element_type=f32) +
                     b1cat[...].astype(f32)).astype(bf16)
    gx = jnp.dot(gates.astype(bf16), expand_ref[...],
                 preferred_element_type=f32).astype(bf16)
    wout = (jnp.dot(hh * gx, w2v[...], preferred_element_type=f32) +
            jnp.dot(gates, eb2x[...], preferred_element_type=f32))
    w_ref[...] = wout

    score = jnp.sum(c_ref[...] * wout, axis=1, keepdims=True)
    prob_ref[...] = 1.0 / (1.0 + jnp.exp(-score))


def _fused(x, c, enc_W0, enc_b0, enc_W1, enc_b1, enc_W2, enc_b2,
           dec_W0, dec_b0, dec_W1, dec_b1, dec_W2b, dec_b2,
           w_gate, w1cat, b1cat, w2v, e_b2, eio, expand):
    full = lambda shp: pl.BlockSpec(shp, lambda i: (0, 0))
    tok = lambda d: pl.BlockSpec((_TN, d), lambda i: (i, 0))
    row = lambda b: b.reshape(1, -1)

    return pl.pallas_call(
        _body,
        grid=(_GRID,),
        in_specs=[
            tok(_XD), tok(_CD),
            full((_XD, _HD)), full((1, _HD)), full((_HD, _HD)), full((1, _HD)),
            full((_HD, _HD)), full((1, _HD)),
            full((_HD, _HD)), full((1, _HD)), full((_HD, _HD)), full((1, _HD)),
            full((_HD, _XD)), full((1, _XD)),
            full((_HD, _E)), full((_HD, _E * _HD)), full((1, _E * _HD)),
            full((_E * _HD, _CD)), full((_E, _CD)),
            full((1, _E)), full((_E, _E * _HD)),
        ],
        out_specs=(
            tok(_CD), tok(1), tok(_HD), tok(_XD), tok(_E),
        ),
        out_shape=(
            jax.ShapeDtypeStruct((_N, _CD), jnp.float32),
            jax.ShapeDtypeStruct((_N, 1), jnp.float32),
            jax.ShapeDtypeStruct((_N, _HD), jnp.float32),
            jax.ShapeDtypeStruct((_N, _XD), jnp.float32),
            jax.ShapeDtypeStruct((_N, _E), jnp.float32),
        ),
        compiler_params=pltpu.CompilerParams(
            dimension_semantics=("arbitrary",)),
    )(x, c, enc_W0, row(enc_b0), enc_W1, row(enc_b1), enc_W2, row(enc_b2),
      dec_W0, row(dec_b0), dec_W1, row(dec_b1), dec_W2b, row(dec_b2),
      w_gate, w1cat, b1cat, w2v, e_b2, eio, expand)


def _stats_sc_body(gates_hbm, part_hbm, g_v, p_v):
    f32 = jnp.float32
    wid = lax.axis_index("s") * 2 + lax.axis_index("c")
    base = wid * _TPW
    pltpu.sync_copy(gates_hbm.at[pl.ds(base, _TPW), :], g_v)

    zero = jnp.zeros((_L,), f32)
    one = jnp.ones((_L,), f32)

    def _row(r, acc):
        out = []
        for cc in range(_E // _L):
            g = g_v[r, pl.ds(cc * _L, _L)]
            out.append(acc[cc] + g)
        for cc in range(_E // _L):
            g = g_v[r, pl.ds(cc * _L, _L)]
            out.append(acc[_E // _L + cc] + jnp.where(g > 0.0, one, zero))
        return tuple(out)

    acc = lax.fori_loop(0, _TPW, _row, tuple([zero] * (2 * _E // _L)))
    for cc in range(_E // _L):
        p_v[0, pl.ds(cc * _L, _L)] = acc[cc]
        p_v[1, pl.ds(cc * _L, _L)] = acc[_E // _L + cc]

    pltpu.sync_copy(p_v, part_hbm.at[wid])


def _stats(gates):
    mesh = plsc.VectorSubcoreMesh(core_axis_name="c", subcore_axis_name="s")
    k = functools.partial(
        pl.kernel,
        mesh=mesh,
        out_type=jax.ShapeDtypeStruct((_NW, 2, _E), jnp.float32),
        scratch_types=[
            pltpu.VMEM((_TPW, _E), jnp.float32),
            pltpu.VMEM((2, _E), jnp.float32),
        ],
    )(_stats_sc_body)
    return k(gates)


def _loss_body(part_ref, loss_ref):
    imp = jnp.sum(part_ref[:, 0, :], axis=0, keepdims=True)
    ld = jnp.sum(part_ref[:, 1, :], axis=0, keepdims=True)

    def cv2(v):
        mean = jnp.sum(v) / _E
        var = jnp.sum((v - mean) ** 2) / (_E - 1)
        return var / (mean * mean + 1e-10)

    loss_ref[0, 0] = (cv2(imp) + cv2(ld)) * _LOSS_COEF


def _loss(parts):
    return pl.pallas_call(
        _loss_body,
        out_specs=pl.BlockSpec(memory_space=pltpu.SMEM),
        out_shape=jax.ShapeDtypeStruct((1, 1), jnp.float32),
    )(parts)


def kernel(x, c, enc_W0, enc_b0, enc_W1, enc_b1, enc_W2, enc_b2,
           dec_W0, dec_b0, dec_W1, dec_b1, dec_W2, dec_b2,
           w_gate, e_W1, e_b1, e_W2, e_b2):
    bf16 = jnp.bfloat16
    w1cat = jnp.transpose(e_W1, (1, 0, 2)).reshape(_HD, _E * _HD).astype(bf16)
    b1cat = e_b1.reshape(1, _E * _HD).astype(bf16)
    w2v = e_W2.reshape(_E * _HD, _CD).astype(bf16)
    expand = jnp.kron(jnp.eye(_E, dtype=bf16), jnp.ones((1, _HD), dtype=bf16))
    eio = jnp.arange(_E, dtype=jnp.int32).reshape(1, _E)

    w, prob, z, xhat, gates = _fused(
        x, c, enc_W0, enc_b0, enc_W1, enc_b1, enc_W2, enc_b2,
        dec_W0, dec_b0, dec_W1, dec_b1, dec_W2.astype(bf16), dec_b2,
        w_gate, w1cat, b1cat, w2v, e_b2, eio, expand)
    parts = _stats(gates)
    loss = _loss(parts)
    return (w, prob, z, xhat, loss.reshape(()), gates)


# speedup vs baseline: 1.1657x; 1.0011x over previous
import functools

import jax
import jax.numpy as jnp
from jax import lax
from jax.experimental import pallas as pl
from jax.experimental.pallas import tpu as pltpu
from jax.experimental.pallas import tpu_sc as plsc

_N = 8192
_XD = 2048
_CD = 128
_HD = 64
_E = 64
_TN = 512
_GRID = _N // _TN
_LOSS_COEF = 1e-2

_NW = 32
_TPW = _N // _NW
_L = 16


def _body(x_ref, c_ref,
          ew0, eb0, ew1, eb1, ew2, eb2,
          dw0, db0, dw1, db1, dw2, db2,
          wg, w1cat, b1cat, w2v, eb2x, eio_ref, expand_ref,
          w_ref, prob_ref, z_ref, xhat_ref, gates_ref):
    f32 = jnp.float32
    bf16 = jnp.bfloat16

    h0_t = lax.dot_general(ew0[...], x_ref[...],
                           dimension_numbers=(((0,), (1,)), ((), ())),
                           preferred_element_type=f32)
    h = jax.nn.relu(h0_t.T + eb0[...])
    h = jax.nn.relu(jnp.dot(h, ew1[...], preferred_element_type=f32) + eb1[...])
    z = jnp.dot(h, ew2[...], preferred_element_type=f32) + eb2[...]
    z_ref[...] = z

    h = jax.nn.relu(jnp.dot(z, dw0[...], preferred_element_type=f32) + db0[...])
    h = jax.nn.relu(jnp.dot(h, dw1[...], preferred_element_type=f32) + db1[...])
    xhat_ref[...] = (jnp.dot(h.astype(bf16), dw2[...],
                             preferred_element_type=f32) + db2[...])

    logits = jnp.dot(z, wg[...], preferred_element_type=f32)
    eio = eio_ref[...]
    m1 = jnp.max(logits, axis=1, keepdims=True)
    i1 = jnp.min(jnp.where(logits == m1, eio, _E), axis=1, keepdims=True)
    masked = jnp.where(eio == i1, -jnp.inf, logits)
    m2 = jnp.max(masked, axis=1, keepdims=True)
    i2 = jnp.min(jnp.where(masked == m2, eio, _E), axis=1, keepdims=True)
    e2 = jnp.exp(m2 - m1)
    denom = 1.0 + e2
    g1 = 1.0 / denom
    g2 = e2 / denom
    gates = jnp.where(eio == i1, g1, 0.0) + jnp.where(eio == i2, g2, 0.0)
    gates_ref[...] = gates

    hh = jax.nn.relu(jnp.dot(z.astype(bf16), w1cat[...],
                             preferred_element_type=f32) +
                     b1cat[...].astype(f32)).astype(bf16)
    gx = jnp.dot(gates.astype(bf16), expand_ref[...],
                 preferred_element_type=f32).astype(bf16)
    wout = (jnp.dot(hh * gx, w2v[...], preferred_element_type=f32) +
            jnp.dot(gates, eb2x[...], preferred_element_type=f32))
    w_ref[...] = wout

    score = jnp.sum(c_ref[...] * wout, axis=1, keepdims=True)
    prob_ref[...] = 1.0 / (1.0 + jnp.exp(-score))


def _fused(x, c, enc_W0, enc_b0, enc_W1, enc_b1, enc_W2, enc_b2,
           dec_W0, dec_b0, dec_W1, dec_b1, dec_W2b, dec_b2,
           w_gate, w1cat, b1cat, w2v, e_b2, eio, expand):
    full = lambda shp: pl.BlockSpec(shp, lambda i: (0, 0))
    tok = lambda d: pl.BlockSpec((_TN, d), lambda i: (i, 0))
    row = lambda b: b.reshape(1, -1)

    return pl.pallas_call(
        _body,
        grid=(_GRID,),
        in_specs=[
            tok(_XD), tok(_CD),
            full((_XD, _HD)), full((1, _HD)), full((_HD, _HD)), full((1, _HD)),
            full((_HD, _HD)), full((1, _HD)),
            full((_HD, _HD)), full((1, _HD)), full((_HD, _HD)), full((1, _HD)),
            full((_HD, _XD)), full((1, _XD)),
            full((_HD, _E)), full((_HD, _E * _HD)), full((1, _E * _HD)),
            full((_E * _HD, _CD)), full((_E, _CD)),
            full((1, _E)), full((_E, _E * _HD)),
        ],
        out_specs=(
            tok(_CD), tok(1), tok(_HD), tok(_XD), tok(_E),
        ),
        out_shape=(
            jax.ShapeDtypeStruct((_N, _CD), jnp.float32),
            jax.ShapeDtypeStruct((_N, 1), jnp.float32),
            jax.ShapeDtypeStruct((_N, _HD), jnp.float32),
            jax.ShapeDtypeStruct((_N, _XD), jnp.float32),
            jax.ShapeDtypeStruct((_N, _E), jnp.float32),
        ),
        compiler_params=pltpu.CompilerParams(
            dimension_semantics=("arbitrary",)),
    )(x, c, enc_W0, row(enc_b0), enc_W1, row(enc_b1), enc_W2, row(enc_b2),
      dec_W0, row(dec_b0), dec_W1, row(dec_b1), dec_W2b, row(dec_b2),
      w_gate, w1cat, b1cat, w2v, e_b2, eio, expand)


def _stats_sc_body(gates_hbm, part_hbm, g_v, p_v):
    f32 = jnp.float32
    wid = lax.axis_index("s") * 2 + lax.axis_index("c")
    base = wid * _TPW
    pltpu.sync_copy(gates_hbm.at[pl.ds(base, _TPW), :], g_v)

    zero = jnp.zeros((_L,), f32)
    one = jnp.ones((_L,), f32)

    def _row(r, acc):
        out = list(acc)
        for u in range(2):
            for cc in range(_E // _L):
                g = g_v[r * 2 + u, pl.ds(cc * _L, _L)]
                out[cc] = out[cc] + g
                out[_E // _L + cc] = (out[_E // _L + cc] +
                                      jnp.where(g > 0.0, one, zero))
        return tuple(out)

    acc = lax.fori_loop(0, _TPW // 2, _row, tuple([zero] * (2 * _E // _L)))
    for cc in range(_E // _L):
        p_v[0, pl.ds(cc * _L, _L)] = acc[cc]
        p_v[1, pl.ds(cc * _L, _L)] = acc[_E // _L + cc]

    pltpu.sync_copy(p_v, part_hbm.at[wid])


def _stats(gates):
    mesh = plsc.VectorSubcoreMesh(core_axis_name="c", subcore_axis_name="s")
    k = functools.partial(
        pl.kernel,
        mesh=mesh,
        out_type=jax.ShapeDtypeStruct((_NW, 2, _E), jnp.float32),
        scratch_types=[
            pltpu.VMEM((_TPW, _E), jnp.float32),
            pltpu.VMEM((2, _E), jnp.float32),
        ],
    )(_stats_sc_body)
    return k(gates)


def _loss_body(part_ref, loss_ref):
    imp = jnp.sum(part_ref[:, 0, :], axis=0, keepdims=True)
    ld = jnp.sum(part_ref[:, 1, :], axis=0, keepdims=True)

    def cv2(v):
        mean = jnp.sum(v) / _E
        var = jnp.sum((v - mean) ** 2) / (_E - 1)
        return var / (mean * mean + 1e-10)

    loss_ref[0, 0] = (cv2(imp) + cv2(ld)) * _LOSS_COEF


def _loss(parts):
    return pl.pallas_call(
        _loss_body,
        out_specs=pl.BlockSpec(memory_space=pltpu.SMEM),
        out_shape=jax.ShapeDtypeStruct((1, 1), jnp.float32),
    )(parts)


def kernel(x, c, enc_W0, enc_b0, enc_W1, enc_b1, enc_W2, enc_b2,
           dec_W0, dec_b0, dec_W1, dec_b1, dec_W2, dec_b2,
           w_gate, e_W1, e_b1, e_W2, e_b2):
    bf16 = jnp.bfloat16
    w1cat = jnp.transpose(e_W1, (1, 0, 2)).reshape(_HD, _E * _HD).astype(bf16)
    b1cat = e_b1.reshape(1, _E * _HD).astype(bf16)
    w2v = e_W2.reshape(_E * _HD, _CD).astype(bf16)
    expand = jnp.kron(jnp.eye(_E, dtype=bf16), jnp.ones((1, _HD), dtype=bf16))
    eio = jnp.arange(_E, dtype=jnp.int32).reshape(1, _E)

    w, prob, z, xhat, gates = _fused(
        x, c, enc_W0, enc_b0, enc_W1, enc_b1, enc_W2, enc_b2,
        dec_W0, dec_b0, dec_W1, dec_b1, dec_W2.astype(bf16), dec_b2,
        w_gate, w1cat, b1cat, w2v, e_b2, eio, expand)
    parts = _stats(gates)
    loss = _loss(parts)
    return (w, prob, z, xhat, loss.reshape(()), gates)
